# baseline (speedup 1.0000x reference)
import functools

import jax
import jax.numpy as jnp
from jax import lax
from jax.experimental import pallas as pl
from jax.experimental.pallas import tpu as pltpu
from jax.experimental.pallas import tpu_sc as plsc

_NC = 2
_NS = 16
_NW = _NC * _NS
_CHUNK = 56


@jax.jit
def _sc_embed(tokens, W_E):
    B, T = tokens.shape
    _, D = W_E.shape
    n_per = (B * T) // _NW
    per_row = T // n_per
    sizes = [_CHUNK] * (n_per // _CHUNK)
    if n_per % _CHUNK:
        sizes.append(n_per % _CHUNK)
    offs = [sum(sizes[:j]) for j in range(len(sizes))]
    n = len(sizes)
    mesh = plsc.VectorSubcoreMesh(core_axis_name="c", subcore_axis_name="s")

    @functools.partial(
        pl.kernel,
        out_type=jax.ShapeDtypeStruct((B, T, D), jnp.float32),
        mesh=mesh,
        scratch_types=[
            pltpu.VMEM((n_per,), jnp.int32),
            pltpu.VMEM((_CHUNK, D), jnp.float32),
            pltpu.VMEM((_CHUNK, D), jnp.float32),
            pltpu.SemaphoreType.DMA,
            pltpu.SemaphoreType.DMA,
        ],
    )
    def k(idx_hbm, table_hbm, out_hbm, idx_v, buf0, buf1, sem0, sem1):
        wid = lax.axis_index("s") * _NC + lax.axis_index("c")
        r = wid // per_row
        col = (wid % per_row) * n_per
        pltpu.sync_copy(idx_hbm.at[r].at[pl.ds(col, n_per)], idx_v)
        bufs = (buf0, buf1)
        sems = (sem0, sem1)

        def gather(j):
            b = j % 2
            return pltpu.make_async_copy(
                table_hbm.at[idx_v.at[pl.ds(offs[j], sizes[j])]],
                bufs[b].at[pl.ds(0, sizes[j])],
                sems[b],
            )

        gather(0).start()
        for j in range(n):
            if j + 1 < n:
                gather(j + 1).start()
            gather(j).wait()
            pltpu.sync_copy(
                bufs[j % 2].at[pl.ds(0, sizes[j])],
                out_hbm.at[r].at[pl.ds(col + offs[j], sizes[j])],
            )

    return k(tokens, W_E)


def kernel(tokens, W_E):
    return _sc_embed(tokens.astype(jnp.int32), W_E)


# speedup vs baseline: 1.0152x; 1.0152x over previous
import functools

import jax
import jax.numpy as jnp
from jax import lax
from jax.experimental import pallas as pl
from jax.experimental.pallas import tpu as pltpu
from jax.experimental.pallas import tpu_sc as plsc

_NC = 2
_NS = 16
_NW = _NC * _NS
_CHUNK = 32


@jax.jit
def _sc_embed(tokens, W_E):
    B, T = tokens.shape
    _, D = W_E.shape
    n_per = (B * T) // _NW
    per_row = T // n_per
    n_chunks = n_per // _CHUNK
    mesh = plsc.VectorSubcoreMesh(core_axis_name="c", subcore_axis_name="s")

    @functools.partial(
        pl.kernel,
        out_type=jax.ShapeDtypeStruct((B, T, D), jnp.float32),
        mesh=mesh,
        scratch_types=[
            pltpu.VMEM((n_per,), jnp.int32),
            pltpu.VMEM((_CHUNK, D), jnp.float32),
            pltpu.VMEM((_CHUNK, D), jnp.float32),
            pltpu.SemaphoreType.DMA,
            pltpu.SemaphoreType.DMA,
        ],
    )
    def k(idx_hbm, table_hbm, out_hbm, idx_v, buf0, buf1, sem0, sem1):
        wid = lax.axis_index("s") * _NC + lax.axis_index("c")
        r = wid // per_row
        col = (wid % per_row) * n_per
        pltpu.sync_copy(idx_hbm.at[r].at[pl.ds(col, n_per)], idx_v)

        def gather(g, buf, sem):
            return pltpu.make_async_copy(
                table_hbm.at[idx_v.at[pl.ds(g * _CHUNK, _CHUNK)]], buf, sem
            )

        def out_slice(g):
            return out_hbm.at[r].at[pl.ds(col + g * _CHUNK, _CHUNK)]

        gather(0, buf0, sem0).start()

        def body(i, carry):
            g = i * 2
            gather(g + 1, buf1, sem1).start()
            gather(g, buf0, sem0).wait()
            pltpu.sync_copy(buf0, out_slice(g))

            @pl.when(g + 2 < n_chunks)
            def _():
                gather(g + 2, buf0, sem0).start()

            gather(g + 1, buf1, sem1).wait()
            pltpu.sync_copy(buf1, out_slice(g + 1))
            return carry

        lax.fori_loop(0, n_chunks // 2, body, 0, unroll=False)

    return k(tokens, W_E)


def kernel(tokens, W_E):
    return _sc_embed(tokens.astype(jnp.int32), W_E)
